# Initial kernel scaffold; baseline (speedup 1.0000x reference)
import jax
import jax.numpy as jnp
from jax.experimental import pallas as pl


def kernel(x, edge_index, W, b):
    raise NotImplementedError("write your pallas kernel here")


# speedup vs baseline: 4.8869x; 4.8869x over previous
import functools

import jax
import jax.numpy as jnp
from jax import lax
from jax.experimental import pallas as pl
from jax.experimental.pallas import tpu as pltpu
from jax.experimental.pallas import tpu_sc as plsc

N = 10000
E = 160000
D = 256
H = 128
TILES = 16
B = 128
NBLK = -(-E // (TILES * B))
EPT = NBLK * B
E_PAD = TILES * EPT
ACC_ROWS = 10240
ROWS_PT = ACC_ROWS // TILES
DEG_CHUNK = ACC_ROWS // TILES


def _sc_body(x0_hbm, x1_hbm, src_hbm, dst_hbm,
             s0_hbm, s1_hbm, deg_hbm,
             src_v, dst_v, buf, ones_v, zeros_v,
             acc, deg_sh,
             sem):
    c = lax.axis_index("c")
    s = lax.axis_index("s")

    pltpu.sync_copy(src_hbm.at[s], src_v)
    pltpu.sync_copy(dst_hbm.at[s], dst_v)

    for i in range(B // 16):
        ones_v[pl.ds(i * 16, 16)] = jnp.ones((16,), jnp.float32)
    for i in range(DEG_CHUNK // 16):
        zeros_v[pl.ds(i * 16, 16)] = jnp.zeros((16,), jnp.float32)

    r0 = s * ROWS_PT

    @pl.when(c == 0)
    def _():
        pltpu.sync_copy(x0_hbm.at[pl.ds(r0, ROWS_PT)], acc.at[pl.ds(r0, ROWS_PT)])
        pltpu.sync_copy(zeros_v, deg_sh.at[pl.ds(s * DEG_CHUNK, DEG_CHUNK)])

    @pl.when(c == 1)
    def _():
        pltpu.sync_copy(x1_hbm.at[pl.ds(r0, ROWS_PT)], acc.at[pl.ds(r0, ROWS_PT)])

    plsc.subcore_barrier()

    def edge_loop(x_hbm, with_deg):
        def blk(j, carry):
            pltpu.async_copy(x_hbm.at[src_v.at[j]], buf, sem).wait()
            pltpu.sync_copy(buf, acc.at[dst_v.at[j]], add=True)
            if with_deg:
                pltpu.sync_copy(ones_v, deg_sh.at[dst_v.at[j]], add=True)
            return carry
        lax.fori_loop(0, NBLK, blk, 0)

    @pl.when(c == 0)
    def _():
        edge_loop(x0_hbm, True)

    @pl.when(c == 1)
    def _():
        edge_loop(x1_hbm, False)

    plsc.subcore_barrier()

    @pl.when(c == 0)
    def _():
        pltpu.sync_copy(acc.at[pl.ds(r0, ROWS_PT)], s0_hbm.at[pl.ds(r0, ROWS_PT)])
        pltpu.sync_copy(deg_sh.at[pl.ds(s * DEG_CHUNK, DEG_CHUNK)],
                        deg_hbm.at[pl.ds(s * DEG_CHUNK, DEG_CHUNK)])

    @pl.when(c == 1)
    def _():
        pltpu.sync_copy(acc.at[pl.ds(r0, ROWS_PT)], s1_hbm.at[pl.ds(r0, ROWS_PT)])


_sc_agg = functools.partial(
    pl.kernel,
    out_type=(
        jax.ShapeDtypeStruct((ACC_ROWS, H), jnp.float32),
        jax.ShapeDtypeStruct((ACC_ROWS, H), jnp.float32),
        jax.ShapeDtypeStruct((ACC_ROWS,), jnp.float32),
    ),
    mesh=plsc.VectorSubcoreMesh(core_axis_name="c", subcore_axis_name="s"),
    scratch_types=[
        pltpu.VMEM((NBLK, B), jnp.int32),
        pltpu.VMEM((NBLK, B), jnp.int32),
        pltpu.VMEM((B, H), jnp.float32),
        pltpu.VMEM((B,), jnp.float32),
        pltpu.VMEM((DEG_CHUNK,), jnp.float32),
        pltpu.VMEM_SHARED((ACC_ROWS, H), jnp.float32),
        pltpu.VMEM_SHARED((ACC_ROWS,), jnp.float32),
        pltpu.SemaphoreType.DMA,
    ],
)(_sc_body)


M_BLK = 1000


def _tc_body(s0_ref, s1_ref, deg_ref, w0_ref, w1_ref, b_ref, out_ref):
    acc = jnp.dot(s0_ref[...], w0_ref[...], preferred_element_type=jnp.float32)
    acc = acc + jnp.dot(s1_ref[...], w1_ref[...], preferred_element_type=jnp.float32)
    deg = jnp.maximum(deg_ref[...], 1.0)
    out_ref[...] = acc / deg + b_ref[...]


_tc_linear = pl.pallas_call(
    _tc_body,
    grid=(N // M_BLK,),
    in_specs=[
        pl.BlockSpec((M_BLK, H), lambda i: (i, 0)),
        pl.BlockSpec((M_BLK, H), lambda i: (i, 0)),
        pl.BlockSpec((M_BLK, 1), lambda i: (i, 0)),
        pl.BlockSpec((H, D), lambda i: (0, 0)),
        pl.BlockSpec((H, D), lambda i: (0, 0)),
        pl.BlockSpec((1, D), lambda i: (0, 0)),
    ],
    out_specs=pl.BlockSpec((M_BLK, D), lambda i: (i, 0)),
    out_shape=jax.ShapeDtypeStruct((N, D), jnp.float32),
)


def kernel(x, edge_index, W, b):
    src = edge_index[0]
    dst = edge_index[1]
    pad = E_PAD - E
    srcp = jnp.concatenate([src, jnp.zeros((pad,), jnp.int32)]).reshape(TILES, NBLK, B)
    dstp = jnp.concatenate([dst, jnp.full((pad,), N, jnp.int32)]).reshape(TILES, NBLK, B)
    xp = jnp.pad(x, ((0, ACC_ROWS - N), (0, 0)))
    s0, s1, deg = _sc_agg(xp[:, :H], xp[:, H:], srcp, dstp)
    s0, s1 = s0[:N], s1[:N]
    w0t = W[:, :H].T
    w1t = W[:, H:].T
    return _tc_linear(s0, s1, deg[:N].reshape(N, 1), w0t, w1t, b.reshape(1, D))
